# baseline (speedup 1.0000x reference)
import functools

import jax
import jax.numpy as jnp
from jax.experimental import pallas as pl


def _fused_body(adj_ref, x_full_ref, x_row_ref, z_ref, w_ref, out_ref, *, k_dim):
    hi = jnp.dot(adj_ref[...], x_full_ref[...], preferred_element_type=jnp.float32)
    acc = x_row_ref[...]
    z = z_ref[...]
    for k in range(k_dim):
        yk = jnp.dot(hi, w_ref[k], preferred_element_type=jnp.float32)
        acc = acc + z[:, k : k + 1] * yk
    out_ref[...] = acc


@jax.jit
def kernel(input, adj, h0, z, weights):
    del h0
    n, f_in = input.shape
    k_dim, _, f_out = weights.shape

    br = 200
    while n % br != 0:
        br //= 2
    grid = (n // br,)

    body = functools.partial(_fused_body, k_dim=k_dim)
    return pl.pallas_call(
        body,
        grid=grid,
        in_specs=[
            pl.BlockSpec((br, n), lambda i: (i, 0)),
            pl.BlockSpec((n, f_in), lambda i: (0, 0)),
            pl.BlockSpec((br, f_in), lambda i: (i, 0)),
            pl.BlockSpec((br, k_dim), lambda i: (i, 0)),
            pl.BlockSpec((k_dim, f_in, f_out), lambda i: (0, 0, 0)),
        ],
        out_specs=pl.BlockSpec((br, f_out), lambda i: (i, 0)),
        out_shape=jax.ShapeDtypeStruct((n, f_out), jnp.float32),
    )(adj, input, input, z, weights)


# speedup vs baseline: 1.0813x; 1.0813x over previous
import functools

import jax
import jax.numpy as jnp
from jax.experimental import pallas as pl


def _fused_body(adj_ref, x_full_ref, x_row_ref, z_ref, w_ref, out_ref, *, k_dim):
    hi = jnp.dot(adj_ref[...], x_full_ref[...], preferred_element_type=jnp.float32)
    acc = x_row_ref[...]
    z = z_ref[...]
    for k in range(k_dim):
        yk = jnp.dot(hi, w_ref[k], preferred_element_type=jnp.float32)
        acc = acc + z[:, k : k + 1] * yk
    out_ref[...] = acc


@jax.jit
def kernel(input, adj, h0, z, weights):
    del h0
    n, f_in = input.shape
    k_dim, _, f_out = weights.shape

    br = 400
    while n % br != 0:
        br //= 2
    grid = (n // br,)

    body = functools.partial(_fused_body, k_dim=k_dim)
    return pl.pallas_call(
        body,
        grid=grid,
        in_specs=[
            pl.BlockSpec((br, n), lambda i: (i, 0)),
            pl.BlockSpec((n, f_in), lambda i: (0, 0)),
            pl.BlockSpec((br, f_in), lambda i: (i, 0)),
            pl.BlockSpec((br, k_dim), lambda i: (i, 0)),
            pl.BlockSpec((k_dim, f_in, f_out), lambda i: (0, 0, 0)),
        ],
        out_specs=pl.BlockSpec((br, f_out), lambda i: (i, 0)),
        out_shape=jax.ShapeDtypeStruct((n, f_out), jnp.float32),
    )(adj, input, input, z, weights)
